# baseline (speedup 1.0000x reference)
import jax
import jax.numpy as jnp
from jax import lax
from jax.experimental import pallas as pl
from jax.experimental.pallas import tpu as pltpu

VOCAB = 100000
OBJ = 100000
DIM = 200
N_NEG = 100
N_IDX = N_NEG + 1
LANE = 128


def _tc_body(cols, word, o0, o1, o2, o3, w_blk, res, wout):
    i = pl.program_id(0)
    lane = lax.rem(cols[i], LANE)

    w = w_blk[pl.ds(lax.rem(word[0], 8), 1), :]
    num_full = jnp.zeros((1, LANE), jnp.float32)
    sq_full = jnp.zeros((1, LANE), jnp.float32)
    off = 0
    for o_ref in (o0, o1, o2, o3):
        o = o_ref[...]
        rows = o.shape[0]
        num_full = num_full + jnp.dot(
            w[:, off:off + rows], o, preferred_element_type=jnp.float32,
            precision=lax.Precision.HIGHEST)
        sq_full = sq_full + jnp.sum(o * o, axis=0, keepdims=True)
        off += rows
    wsq = jnp.sum(w * w)

    eps2 = jnp.float32(1e-16)
    denom2 = jnp.maximum(wsq, eps2) * jnp.maximum(sq_full, eps2)
    r_vec = num_full * lax.rsqrt(denom2)

    lane_iota = lax.broadcasted_iota(jnp.int32, (1, LANE), 1)
    r_scalar = jnp.sum(jnp.where(lane_iota == lane, r_vec, 0.0))

    @pl.when(i == 0)
    def _():
        res[...] = jnp.zeros((1, LANE), jnp.float32)
        wout[...] = w
    res[...] = jnp.where(lane_iota == i, r_scalar, res[...])


_grid_spec = pltpu.PrefetchScalarGridSpec(
    num_scalar_prefetch=2,
    grid=(N_IDX,),
    in_specs=[
        pl.BlockSpec((64, LANE), lambda i, cols, word: (0, cols[i] // LANE)),
        pl.BlockSpec((64, LANE), lambda i, cols, word: (1, cols[i] // LANE)),
        pl.BlockSpec((64, LANE), lambda i, cols, word: (2, cols[i] // LANE)),
        pl.BlockSpec((8, LANE), lambda i, cols, word: (24, cols[i] // LANE)),
        pl.BlockSpec((8, DIM), lambda i, cols, word: (word[0] // 8, 0)),
    ],
    out_specs=[
        pl.BlockSpec((1, LANE), lambda i, cols, word: (0, 0)),
        pl.BlockSpec((1, DIM), lambda i, cols, word: (0, 0)),
    ],
)

_tc_call = pl.pallas_call(
    _tc_body,
    grid_spec=_grid_spec,
    out_shape=(
        jax.ShapeDtypeStruct((1, LANE), jnp.float32),
        jax.ShapeDtypeStruct((1, DIM), jnp.float32),
    ),
)


def kernel(W, O, word, obj, neg_samples):
    word = jnp.asarray(word, jnp.int32).reshape(1)
    obj = jnp.asarray(obj, jnp.int32)
    neg = jnp.asarray(neg_samples, jnp.int32)
    cols = jnp.concatenate([obj.reshape(1), neg])

    res, wout = _tc_call(cols, word, O, O, O, O, W)
    word_embed = wout
    obj_embed = res[0, 0]
    neg_embeds = res[0, 1:1 + N_NEG]
    return (word_embed, obj_embed, neg_embeds)
